# baseline (speedup 1.0000x reference)
import functools

import jax
import jax.numpy as jnp
from jax.experimental import pallas as pl
from jax.experimental.pallas import tpu as pltpu

_WORLD_SIZE = 8
_RANK = 0
_MARGIN = 1.0
_COL_BLOCK = 2048
_SUB_BLOCK = 512
_LANES = 128
_EXT = 16


def _triplet_body(x_ref, y_ref, out_ref, xs_ref, ye_ref, pos_ref, neg_ref, *,
                  rows, edim, cluster_size, n_col_blocks):
    j = pl.program_id(0)
    inf = jnp.float32(jnp.inf)
    n_sub = _COL_BLOCK // _SUB_BLOCK
    n_chunk = _SUB_BLOCK // _LANES

    @pl.when(j == 0)
    def _():
        xs_ref[:, 0:edim] = (x_ref[...] * -2.0).astype(jnp.bfloat16)
        one_col = jax.lax.broadcasted_iota(jnp.int32, (rows, _EXT), 1) == 0
        xs_ref[:, edim:edim + _EXT] = jnp.where(
            one_col, 1.0, 0.0).astype(jnp.bfloat16)
        neg_ref[...] = jnp.full((rows, _LANES), inf, jnp.float32)

    y16 = y_ref[...].astype(jnp.bfloat16)
    ye_ref[:, 0:edim] = y16
    yq = y16.astype(jnp.float32)
    y2col = jax.lax.dot_general(
        yq * yq, jnp.ones((edim, _EXT), jnp.float32),
        (((1,), (0,)), ((), ())), preferred_element_type=jnp.float32,
    )
    col0 = jax.lax.broadcasted_iota(jnp.int32, (_COL_BLOCK, _EXT), 1) == 0
    ye_ref[:, edim:edim + _EXT] = jnp.where(
        col0, y2col, 0.0).astype(jnp.bfloat16)

    def sweep(band_cols):
        if band_cols:
            li = jax.lax.broadcasted_iota(
                jnp.int32, (_LANES, _LANES), 0) // cluster_size
            ci = jax.lax.broadcasted_iota(
                jnp.int32, (_LANES, _LANES), 1) // cluster_size
            bm = li == ci
            inf_mask = jnp.where(bm, inf, 0.0)
            ninf_mask = jnp.where(bm, 0.0, -inf)
        pend = []
        for s in range(n_sub):
            base = s * _SUB_BLOCK
            ycb = ye_ref[base:base + _SUB_BLOCK, :]
            m = jax.lax.dot_general(
                xs_ref[...], ycb, (((1,), (1,)), ((), ())),
                preferred_element_type=jnp.float32,
            )
            for k in range(n_chunk):
                col = base + k * _LANES
                chunk = m[:, k * _LANES:(k + 1) * _LANES]
                if col < band_cols:
                    r0 = col
                    if r0 > 0:
                        neg_ref[0:r0, :] = jnp.minimum(
                            neg_ref[0:r0, :], chunk[0:r0, :])
                    if r0 + _LANES < rows:
                        neg_ref[r0 + _LANES:rows, :] = jnp.minimum(
                            neg_ref[r0 + _LANES:rows, :],
                            chunk[r0 + _LANES:rows, :])
                    sub = chunk[r0:r0 + _LANES, :]
                    neg_ref[r0:r0 + _LANES, :] = jnp.minimum(
                        neg_ref[r0:r0 + _LANES, :], sub + inf_mask)
                    pos_ref[r0:r0 + _LANES, :] = sub + ninf_mask
                else:
                    pend.append(chunk)
        while len(pend) > 1:
            pend = [jnp.minimum(pend[i], pend[i + 1])
                    if i + 1 < len(pend) else pend[i]
                    for i in range(0, len(pend), 2)]
        if pend:
            neg_ref[...] = jnp.minimum(neg_ref[...], pend[0])

    @pl.when(j == 0)
    def _():
        sweep(min(rows, _COL_BLOCK))

    @pl.when(j > 0)
    def _():
        sweep(0)

    @pl.when(j == n_col_blocks - 1)
    def _():
        x = xs_ref[:, 0:edim].astype(jnp.float32) * -0.5
        xx = x * x
        x2m = jax.lax.dot_general(
            xx, jnp.ones((edim, _LANES), jnp.float32),
            (((1,), (0,)), ((), ())), preferred_element_type=jnp.float32,
        )
        x2 = x2m[:, 0:1]
        pos = jnp.max(pos_ref[...], axis=1, keepdims=True)
        neg = jnp.min(neg_ref[...], axis=1, keepdims=True)
        ap = jnp.sqrt(jnp.maximum(x2 + pos, 0.0))
        an = jnp.sqrt(jnp.maximum(x2 + neg, 0.0))
        out_ref[...] = jnp.sum(jnp.maximum(ap - an + _MARGIN, 0.0)).reshape(1, 1)


def kernel(batch):
    n_clusters, cluster_size, edim = batch.shape
    all_embeds = batch.reshape(-1, edim)
    n_total = all_embeds.shape[0]
    base, rem = divmod(n_clusters, _WORLD_SIZE)
    my_clusters = base + (1 if _RANK < rem else 0)
    my_rows = my_clusters * cluster_size
    n_col_blocks = n_total // _COL_BLOCK

    out = pl.pallas_call(
        functools.partial(_triplet_body, rows=my_rows, edim=edim,
                          cluster_size=cluster_size,
                          n_col_blocks=n_col_blocks),
        grid=(n_col_blocks,),
        in_specs=[
            pl.BlockSpec((my_rows, edim), lambda j: (0, 0)),
            pl.BlockSpec((_COL_BLOCK, edim), lambda j: (j, 0)),
        ],
        out_specs=pl.BlockSpec((1, 1), lambda j: (0, 0)),
        out_shape=jax.ShapeDtypeStruct((1, 1), jnp.float32),
        scratch_shapes=[
            pltpu.VMEM((my_rows, edim + _EXT), jnp.bfloat16),
            pltpu.VMEM((_COL_BLOCK, edim + _EXT), jnp.bfloat16),
            pltpu.VMEM((my_rows, _LANES), jnp.float32),
            pltpu.VMEM((my_rows, _LANES), jnp.float32),
        ],
        compiler_params=pltpu.CompilerParams(
            dimension_semantics=("arbitrary",),
        ),
    )(all_embeds, all_embeds)
    return out[0, 0] / my_rows


# speedup vs baseline: 1.0751x; 1.0751x over previous
import functools

import jax
import jax.numpy as jnp
from jax.experimental import pallas as pl
from jax.experimental.pallas import tpu as pltpu

_WORLD_SIZE = 8
_RANK = 0
_MARGIN = 1.0
_COL_BLOCK = 4096
_SUB_BLOCK = 512
_LANES = 128


def _triplet_body(x_ref, y_ref, out_ref, xs_ref, pos_ref, neg_ref, *,
                  rows, edim, cluster_size, n_col_blocks):
    j = pl.program_id(0)
    inf = jnp.float32(jnp.inf)
    n_sub = _COL_BLOCK // _SUB_BLOCK
    n_chunk = _SUB_BLOCK // _LANES

    @pl.when(j == 0)
    def _():
        xs_ref[...] = x_ref[...] * -2.0
        neg_ref[...] = jnp.full((rows, _LANES), inf, jnp.float32)

    y = y_ref[...]
    y2blk = jax.lax.dot_general(
        jnp.ones((8, edim), jnp.float32), y * y,
        (((1,), (1,)), ((), ())), preferred_element_type=jnp.float32,
    )

    def sweep(band_cols):
        if band_cols:
            li = jax.lax.broadcasted_iota(
                jnp.int32, (_LANES, _LANES), 0) // cluster_size
            ci = jax.lax.broadcasted_iota(
                jnp.int32, (_LANES, _LANES), 1) // cluster_size
            bm = li == ci
            inf_mask = jnp.where(bm, inf, 0.0)
            ninf_mask = jnp.where(bm, 0.0, -inf)
        pend = []
        for s in range(n_sub):
            base = s * _SUB_BLOCK
            ycb = y_ref[base:base + _SUB_BLOCK, :]
            m = jax.lax.dot_general(
                xs_ref[...], ycb, (((1,), (1,)), ((), ())),
                preferred_element_type=jnp.float32,
            )
            for k in range(n_chunk):
                col = base + k * _LANES
                y2c = y2blk[0:1, col:col + _LANES]
                chunk = m[:, k * _LANES:(k + 1) * _LANES] + y2c
                if col < band_cols:
                    r0 = col
                    if r0 > 0:
                        neg_ref[0:r0, :] = jnp.minimum(
                            neg_ref[0:r0, :], chunk[0:r0, :])
                    if r0 + _LANES < rows:
                        neg_ref[r0 + _LANES:rows, :] = jnp.minimum(
                            neg_ref[r0 + _LANES:rows, :],
                            chunk[r0 + _LANES:rows, :])
                    sub = chunk[r0:r0 + _LANES, :]
                    neg_ref[r0:r0 + _LANES, :] = jnp.minimum(
                        neg_ref[r0:r0 + _LANES, :], sub + inf_mask)
                    pos_ref[r0:r0 + _LANES, :] = sub + ninf_mask
                else:
                    pend.append(chunk)
        while len(pend) > 1:
            pend = [jnp.minimum(pend[i], pend[i + 1])
                    if i + 1 < len(pend) else pend[i]
                    for i in range(0, len(pend), 2)]
        if pend:
            neg_ref[...] = jnp.minimum(neg_ref[...], pend[0])

    @pl.when(j == 0)
    def _():
        sweep(min(rows, _COL_BLOCK))

    @pl.when(j > 0)
    def _():
        sweep(0)

    @pl.when(j == n_col_blocks - 1)
    def _():
        x = x_ref[...]
        xx = x * x
        x2m = jax.lax.dot_general(
            xx, jnp.ones((edim, _LANES), jnp.float32),
            (((1,), (0,)), ((), ())), preferred_element_type=jnp.float32,
        )
        x2 = x2m[:, 0:1]
        pos = jnp.max(pos_ref[...], axis=1, keepdims=True)
        neg = jnp.min(neg_ref[...], axis=1, keepdims=True)
        ap = jnp.sqrt(jnp.maximum(x2 + pos, 0.0))
        an = jnp.sqrt(jnp.maximum(x2 + neg, 0.0))
        out_ref[...] = jnp.sum(jnp.maximum(ap - an + _MARGIN, 0.0)).reshape(1, 1)


def kernel(batch):
    n_clusters, cluster_size, edim = batch.shape
    all_embeds = batch.reshape(-1, edim)
    n_total = all_embeds.shape[0]
    base, rem = divmod(n_clusters, _WORLD_SIZE)
    my_clusters = base + (1 if _RANK < rem else 0)
    my_rows = my_clusters * cluster_size
    n_col_blocks = n_total // _COL_BLOCK

    out = pl.pallas_call(
        functools.partial(_triplet_body, rows=my_rows, edim=edim,
                          cluster_size=cluster_size,
                          n_col_blocks=n_col_blocks),
        grid=(n_col_blocks,),
        in_specs=[
            pl.BlockSpec((my_rows, edim), lambda j: (0, 0)),
            pl.BlockSpec((_COL_BLOCK, edim), lambda j: (j, 0)),
        ],
        out_specs=pl.BlockSpec((1, 1), lambda j: (0, 0)),
        out_shape=jax.ShapeDtypeStruct((1, 1), jnp.float32),
        scratch_shapes=[
            pltpu.VMEM((my_rows, edim), jnp.float32),
            pltpu.VMEM((my_rows, _LANES), jnp.float32),
            pltpu.VMEM((my_rows, _LANES), jnp.float32),
        ],
        compiler_params=pltpu.CompilerParams(
            dimension_semantics=("arbitrary",),
        ),
    )(all_embeds, all_embeds)
    return out[0, 0] / my_rows


# speedup vs baseline: 1.1184x; 1.0402x over previous
import functools

import jax
import jax.numpy as jnp
from jax.experimental import pallas as pl
from jax.experimental.pallas import tpu as pltpu

_WORLD_SIZE = 8
_RANK = 0
_MARGIN = 1.0
_COL_BLOCK = 8192
_SUB_BLOCK = 512
_LANES = 128


def _triplet_body(x_ref, y_ref, out_ref, xs_ref, pos_ref, neg_ref, *,
                  rows, edim, cluster_size, n_col_blocks):
    j = pl.program_id(0)
    inf = jnp.float32(jnp.inf)
    n_sub = _COL_BLOCK // _SUB_BLOCK
    n_chunk = _SUB_BLOCK // _LANES

    @pl.when(j == 0)
    def _():
        xs_ref[...] = x_ref[...] * -2.0
        neg_ref[...] = jnp.full((rows, _LANES), inf, jnp.float32)

    y = y_ref[...]
    y2blk = jax.lax.dot_general(
        jnp.ones((8, edim), jnp.float32), y * y,
        (((1,), (1,)), ((), ())), preferred_element_type=jnp.float32,
    )

    def sweep(band_cols):
        if band_cols:
            li = jax.lax.broadcasted_iota(
                jnp.int32, (_LANES, _LANES), 0) // cluster_size
            ci = jax.lax.broadcasted_iota(
                jnp.int32, (_LANES, _LANES), 1) // cluster_size
            bm = li == ci
            inf_mask = jnp.where(bm, inf, 0.0)
            ninf_mask = jnp.where(bm, 0.0, -inf)
        pend = []
        for s in range(n_sub):
            base = s * _SUB_BLOCK
            ycb = y_ref[base:base + _SUB_BLOCK, :]
            m = jax.lax.dot_general(
                xs_ref[...], ycb, (((1,), (1,)), ((), ())),
                preferred_element_type=jnp.float32,
            )
            for k in range(n_chunk):
                col = base + k * _LANES
                y2c = y2blk[0:1, col:col + _LANES]
                chunk = m[:, k * _LANES:(k + 1) * _LANES] + y2c
                if col < band_cols:
                    r0 = col
                    if r0 > 0:
                        neg_ref[0:r0, :] = jnp.minimum(
                            neg_ref[0:r0, :], chunk[0:r0, :])
                    if r0 + _LANES < rows:
                        neg_ref[r0 + _LANES:rows, :] = jnp.minimum(
                            neg_ref[r0 + _LANES:rows, :],
                            chunk[r0 + _LANES:rows, :])
                    sub = chunk[r0:r0 + _LANES, :]
                    neg_ref[r0:r0 + _LANES, :] = jnp.minimum(
                        neg_ref[r0:r0 + _LANES, :], sub + inf_mask)
                    pos_ref[r0:r0 + _LANES, :] = sub + ninf_mask
                else:
                    pend.append(chunk)
        while len(pend) > 1:
            pend = [jnp.minimum(pend[i], pend[i + 1])
                    if i + 1 < len(pend) else pend[i]
                    for i in range(0, len(pend), 2)]
        if pend:
            neg_ref[...] = jnp.minimum(neg_ref[...], pend[0])

    @pl.when(j == 0)
    def _():
        sweep(min(rows, _COL_BLOCK))

    @pl.when(j > 0)
    def _():
        sweep(0)

    @pl.when(j == n_col_blocks - 1)
    def _():
        x = x_ref[...]
        xx = x * x
        x2m = jax.lax.dot_general(
            xx, jnp.ones((edim, _LANES), jnp.float32),
            (((1,), (0,)), ((), ())), preferred_element_type=jnp.float32,
        )
        x2 = x2m[:, 0:1]
        pos = jnp.max(pos_ref[...], axis=1, keepdims=True)
        neg = jnp.min(neg_ref[...], axis=1, keepdims=True)
        ap = jnp.sqrt(jnp.maximum(x2 + pos, 0.0))
        an = jnp.sqrt(jnp.maximum(x2 + neg, 0.0))
        out_ref[...] = jnp.sum(jnp.maximum(ap - an + _MARGIN, 0.0)).reshape(1, 1)


def kernel(batch):
    n_clusters, cluster_size, edim = batch.shape
    all_embeds = batch.reshape(-1, edim)
    n_total = all_embeds.shape[0]
    base, rem = divmod(n_clusters, _WORLD_SIZE)
    my_clusters = base + (1 if _RANK < rem else 0)
    my_rows = my_clusters * cluster_size
    n_col_blocks = n_total // _COL_BLOCK

    out = pl.pallas_call(
        functools.partial(_triplet_body, rows=my_rows, edim=edim,
                          cluster_size=cluster_size,
                          n_col_blocks=n_col_blocks),
        grid=(n_col_blocks,),
        in_specs=[
            pl.BlockSpec((my_rows, edim), lambda j: (0, 0)),
            pl.BlockSpec((_COL_BLOCK, edim), lambda j: (j, 0)),
        ],
        out_specs=pl.BlockSpec((1, 1), lambda j: (0, 0)),
        out_shape=jax.ShapeDtypeStruct((1, 1), jnp.float32),
        scratch_shapes=[
            pltpu.VMEM((my_rows, edim), jnp.float32),
            pltpu.VMEM((my_rows, _LANES), jnp.float32),
            pltpu.VMEM((my_rows, _LANES), jnp.float32),
        ],
        compiler_params=pltpu.CompilerParams(
            dimension_semantics=("arbitrary",),
        ),
    )(all_embeds, all_embeds)
    return out[0, 0] / my_rows
